# Initial kernel scaffold; baseline (speedup 1.0000x reference)
import jax
import jax.numpy as jnp
from jax.experimental import pallas as pl


def kernel(x, edge_index, edge_features, W1, b1, W2, b2, Wih0, Whh0, bih0, bhh0, Wih1, Whh1, bih1, bhh1, We, be, Wc, bc):
    raise NotImplementedError("write your pallas kernel here")


# speedup vs baseline: 14.3946x; 14.3946x over previous
import functools

import jax
import jax.numpy as jnp
from jax import lax
from jax.experimental import pallas as pl
from jax.experimental.pallas import tpu as pltpu
from jax.experimental.pallas import tpu_sc as plsc

N = 10000
E = 320000
D = 128
H = 128
G4 = 4 * H
NW = 32
NPAD = 10240
RPT = NPAD // 16
K = 128
NCHUNK = E // K
CPT = (NCHUNK + NW - 1) // NW
BN = 1000
BE = 4000
CSEQ = 1000
NT = N + CSEQ

_mesh = plsc.VectorSubcoreMesh(
    core_axis_name="c", subcore_axis_name="s", num_cores=2, num_subcores=16)


@functools.partial(
    pl.kernel,
    out_type=jax.ShapeDtypeStruct((2, NPAD, D), jnp.float32),
    mesh=_mesh,
    scratch_types=[
        pltpu.VMEM_SHARED((NPAD, D), jnp.float32),
        pltpu.VMEM((K, D), jnp.float32),
        pltpu.VMEM((K,), jnp.int32),
    ],
)
def _sc_degree(dst_hbm, zrows_hbm, orows_hbm, out_hbm, acc, rows, didx):
    c = lax.axis_index("c")
    s = lax.axis_index("s")
    wid = s * 2 + c
    pltpu.sync_copy(zrows_hbm, rows)
    for j in range(RPT // K):
        pltpu.sync_copy(rows, acc.at[pl.ds(s * RPT + j * K, K)])
    pltpu.sync_copy(orows_hbm, rows)
    plsc.subcore_barrier()

    def chunk(k, carry):
        cid = wid + k * NW

        @pl.when(cid < NCHUNK)
        def _():
            pltpu.sync_copy(dst_hbm.at[pl.ds(cid * K, K)], didx)
            pltpu.sync_copy(rows, acc.at[didx], add=True)

        return carry

    lax.fori_loop(0, CPT, chunk, 0)
    plsc.subcore_barrier()
    for j in range(RPT // K):
        pltpu.sync_copy(acc.at[pl.ds(s * RPT + j * K, K)], rows)
        pltpu.sync_copy(rows, out_hbm.at[c, pl.ds(s * RPT + j * K, K)])


@functools.partial(
    pl.kernel,
    out_type=jax.ShapeDtypeStruct((2, NPAD, D), jnp.float32),
    mesh=_mesh,
    scratch_types=[
        pltpu.VMEM_SHARED((NPAD, D), jnp.float32),
        pltpu.VMEM((K, D), jnp.float32),
        pltpu.VMEM((K,), jnp.int32),
        pltpu.VMEM((K,), jnp.int32),
        pltpu.SemaphoreType.DMA,
    ],
)
def _sc_msg(g_hbm, src_hbm, dst_hbm, zrows_hbm, out_hbm, acc, rows, sidx, didx, sem):
    c = lax.axis_index("c")
    s = lax.axis_index("s")
    wid = s * 2 + c
    pltpu.sync_copy(zrows_hbm, rows)
    for j in range(RPT // K):
        pltpu.sync_copy(rows, acc.at[pl.ds(s * RPT + j * K, K)])
    plsc.subcore_barrier()

    def chunk(k, carry):
        cid = wid + k * NW

        @pl.when(cid < NCHUNK)
        def _():
            b = cid * K
            pltpu.sync_copy(src_hbm.at[pl.ds(b, K)], sidx)
            pltpu.sync_copy(dst_hbm.at[pl.ds(b, K)], didx)
            pltpu.async_copy(g_hbm.at[sidx], rows, sem).wait()
            pltpu.sync_copy(rows, acc.at[didx], add=True)

        return carry

    lax.fori_loop(0, CPT, chunk, 0)
    plsc.subcore_barrier()
    for j in range(RPT // K):
        pltpu.sync_copy(acc.at[pl.ds(s * RPT + j * K, K)], rows)
        pltpu.sync_copy(rows, out_hbm.at[c, pl.ds(s * RPT + j * K, K)])


@functools.partial(
    pl.kernel,
    out_type=[
        jax.ShapeDtypeStruct((E, D), jnp.float32),
        jax.ShapeDtypeStruct((E, D), jnp.float32),
        jax.ShapeDtypeStruct((E,), jnp.float32),
    ],
    mesh=_mesh,
    scratch_types=[
        pltpu.VMEM((K, D), jnp.float32),
        pltpu.VMEM((K, D), jnp.float32),
        pltpu.VMEM((K,), jnp.int32),
        pltpu.VMEM((K,), jnp.int32),
        pltpu.VMEM((K,), jnp.float32),
        pltpu.VMEM((K,), jnp.float32),
        pltpu.VMEM((K,), jnp.float32),
        pltpu.VMEM((K,), jnp.float32),
        pltpu.SemaphoreType.DMA,
    ],
)
def _sc_edges(ys_hbm, src_hbm, dst_hbm, sa_hbm, sb_hbm, et_hbm,
              srco_hbm, dsto_hbm, pred_hbm,
              bufs, bufd, sidx, didx, et_v, pa_v, pb_v, pr_v, sem):
    c = lax.axis_index("c")
    s = lax.axis_index("s")
    wid = s * 2 + c

    def chunk(k, carry):
        cid = wid + k * NW

        @pl.when(cid < NCHUNK)
        def _():
            b = cid * K
            pltpu.sync_copy(src_hbm.at[pl.ds(b, K)], sidx)
            pltpu.sync_copy(dst_hbm.at[pl.ds(b, K)], didx)
            pltpu.sync_copy(et_hbm.at[pl.ds(b, K)], et_v)
            pltpu.async_copy(ys_hbm.at[sidx], bufs, sem).wait()
            pltpu.async_copy(ys_hbm.at[didx], bufd, sem).wait()
            pltpu.sync_copy(bufs, srco_hbm.at[pl.ds(b, K)])
            pltpu.sync_copy(bufd, dsto_hbm.at[pl.ds(b, K)])
            pltpu.async_copy(sa_hbm.at[sidx], pa_v, sem).wait()
            pltpu.async_copy(sb_hbm.at[didx], pb_v, sem).wait()
            for j in range(K // 16):
                sl = pl.ds(j * 16, 16)
                pr_v[sl] = pa_v[sl] + pb_v[sl] + et_v[sl]
            pltpu.sync_copy(pr_v, pred_hbm.at[pl.ds(b, K)])

        return carry

    lax.fori_loop(0, CPT, chunk, 0)


def _mm_scale_body(x_ref, w_ref, d_ref, o_ref):
    o_ref[...] = jnp.dot(x_ref[...], w_ref[...],
                         preferred_element_type=jnp.float32) * d_ref[...]


def _mm_scale(x, w, d2):
    return pl.pallas_call(
        _mm_scale_body,
        grid=(N // BN,),
        in_specs=[
            pl.BlockSpec((BN, D), lambda i: (i, 0)),
            pl.BlockSpec((D, D), lambda i: (0, 0)),
            pl.BlockSpec((BN, 1), lambda i: (i, 0)),
        ],
        out_specs=pl.BlockSpec((BN, D), lambda i: (i, 0)),
        out_shape=jax.ShapeDtypeStruct((N, D), jnp.float32),
    )(x, w, d2)


def _gcn2_body(a0_ref, a1_ref, g_ref, d_ref, b_ref, w_ref, o_ref):
    h = d_ref[...] * (a0_ref[...] + a1_ref[...] + g_ref[...]) + b_ref[...]
    h = jnp.maximum(h, 0.0)
    o_ref[...] = jnp.dot(h, w_ref[...],
                         preferred_element_type=jnp.float32) * d_ref[...]


def _gcn2(a0, a1, g, d2, b, w):
    return pl.pallas_call(
        _gcn2_body,
        grid=(N // BN,),
        in_specs=[
            pl.BlockSpec((BN, D), lambda i: (i, 0)),
            pl.BlockSpec((BN, D), lambda i: (i, 0)),
            pl.BlockSpec((BN, D), lambda i: (i, 0)),
            pl.BlockSpec((BN, 1), lambda i: (i, 0)),
            pl.BlockSpec((1, D), lambda i: (0, 0)),
            pl.BlockSpec((D, D), lambda i: (0, 0)),
        ],
        out_specs=pl.BlockSpec((BN, D), lambda i: (i, 0)),
        out_shape=jax.ShapeDtypeStruct((N, D), jnp.float32),
    )(a0, a1, g, d2, b, w)


def _gcn3_body(a0_ref, a1_ref, g_ref, d_ref, b_ref, wT_ref, bs_ref, o_ref):
    h = d_ref[...] * (a0_ref[...] + a1_ref[...] + g_ref[...]) + b_ref[...]
    o_ref[...] = jnp.dot(h, wT_ref[...],
                         preferred_element_type=jnp.float32) + bs_ref[...]


def _gcn3(a0, a1, g, d2, b, wT, bs):
    return pl.pallas_call(
        _gcn3_body,
        grid=(N // BN,),
        in_specs=[
            pl.BlockSpec((BN, D), lambda i: (i, 0)),
            pl.BlockSpec((BN, D), lambda i: (i, 0)),
            pl.BlockSpec((BN, D), lambda i: (i, 0)),
            pl.BlockSpec((BN, 1), lambda i: (i, 0)),
            pl.BlockSpec((1, D), lambda i: (0, 0)),
            pl.BlockSpec((D, G4), lambda i: (0, 0)),
            pl.BlockSpec((1, G4), lambda i: (0, 0)),
        ],
        out_specs=pl.BlockSpec((BN, G4), lambda i: (i, 0)),
        out_shape=jax.ShapeDtypeStruct((N, G4), jnp.float32),
    )(a0, a1, g, d2, b, wT, bs)


def _mm_bias_body(x_ref, w_ref, b_ref, o_ref):
    o_ref[...] = jnp.dot(x_ref[...], w_ref[...],
                         preferred_element_type=jnp.float32) + b_ref[...]


def _mm_bias(x, w, b, rows, bn, kdim, wdim):
    return pl.pallas_call(
        _mm_bias_body,
        grid=(rows // bn,),
        in_specs=[
            pl.BlockSpec((bn, kdim), lambda i: (i, 0)),
            pl.BlockSpec((kdim, wdim), lambda i: (0, 0)),
            pl.BlockSpec((1, wdim), lambda i: (0, 0)),
        ],
        out_specs=pl.BlockSpec((bn, wdim), lambda i: (i, 0)),
        out_shape=jax.ShapeDtypeStruct((rows, wdim), jnp.float32),
    )(x, w, b)


def _lstm2_body(p_ref, m_ref, b1_ref, ys_ref, h0_ref, c0_ref, h1_ref, c1_ref,
                z_s, c0_s, c1_s):
    pid = pl.program_id(0)

    @pl.when(pid == 0)
    def _():
        z_s[...] = jnp.zeros_like(z_s)
        c0_s[...] = jnp.zeros_like(c0_s)
        c1_s[...] = jnp.zeros_like(c1_s)

    def step(i, carry):
        t = pid * CSEQ + i
        z = z_s[...]
        gates = jnp.dot(z, m_ref[...], preferred_element_type=jnp.float32)
        g0 = gates[:, :G4] + p_ref[pl.ds(i, 1), :]
        g1 = gates[:, G4:] + b1_ref[...]
        i0 = jax.nn.sigmoid(g0[:, 0:H])
        f0 = jax.nn.sigmoid(g0[:, H:2 * H])
        gg0 = jnp.tanh(g0[:, 2 * H:3 * H])
        o0 = jax.nn.sigmoid(g0[:, 3 * H:4 * H])
        c0n = f0 * c0_s[...] + i0 * gg0
        h0n = o0 * jnp.tanh(c0n)
        m0 = t < N
        c0_s[...] = jnp.where(m0, c0n, c0_s[...])
        h0u = jnp.where(m0, h0n, z[:, :H])
        i1 = jax.nn.sigmoid(g1[:, 0:H])
        f1 = jax.nn.sigmoid(g1[:, H:2 * H])
        gg1 = jnp.tanh(g1[:, 2 * H:3 * H])
        o1 = jax.nn.sigmoid(g1[:, 3 * H:4 * H])
        c1n = f1 * c1_s[...] + i1 * gg1
        h1n = o1 * jnp.tanh(c1n)
        m1 = jnp.logical_and(t >= 1, t <= N)
        c1_s[...] = jnp.where(m1, c1n, c1_s[...])
        h1u = jnp.where(m1, h1n, z[:, H:])
        z_s[...] = jnp.concatenate([h0u, h1u], axis=1)
        ys_ref[pl.ds(i, 1), :] = h1u
        return carry

    lax.fori_loop(0, CSEQ, step, 0)
    h0_ref[...] = z_s[:, :H]
    h1_ref[...] = z_s[:, H:]
    c0_ref[...] = c0_s[...]
    c1_ref[...] = c1_s[...]


def _lstm2(p0pad, m, bsum1):
    return pl.pallas_call(
        _lstm2_body,
        grid=(NT // CSEQ,),
        in_specs=[
            pl.BlockSpec((CSEQ, G4), lambda i: (i, 0)),
            pl.BlockSpec((2 * H, 2 * G4), lambda i: (0, 0)),
            pl.BlockSpec((1, G4), lambda i: (0, 0)),
        ],
        out_specs=[
            pl.BlockSpec((CSEQ, H), lambda i: (i, 0)),
            pl.BlockSpec((1, H), lambda i: (0, 0)),
            pl.BlockSpec((1, H), lambda i: (0, 0)),
            pl.BlockSpec((1, H), lambda i: (0, 0)),
            pl.BlockSpec((1, H), lambda i: (0, 0)),
        ],
        out_shape=[
            jax.ShapeDtypeStruct((NT, H), jnp.float32),
            jax.ShapeDtypeStruct((1, H), jnp.float32),
            jax.ShapeDtypeStruct((1, H), jnp.float32),
            jax.ShapeDtypeStruct((1, H), jnp.float32),
            jax.ShapeDtypeStruct((1, H), jnp.float32),
        ],
        scratch_shapes=[
            pltpu.VMEM((1, 2 * H), jnp.float32),
            pltpu.VMEM((1, H), jnp.float32),
            pltpu.VMEM((1, H), jnp.float32),
        ],
    )(p0pad, m, bsum1)


def kernel(x, edge_index, edge_features, W1, b1, W2, b2,
           Wih0, Whh0, bih0, bhh0, Wih1, Whh1, bih1, bhh1,
           We, be, Wc, bc):
    f32 = jnp.float32
    src = edge_index[0]
    dst = edge_index[1]

    zrows = jnp.zeros((K, D), f32)
    orows = jnp.ones((K, D), f32)

    dacc = _sc_degree(dst, zrows, orows)
    deg = dacc[0, :N, 0] + dacc[1, :N, 0] + 1.0
    d2 = lax.rsqrt(deg)[:, None]

    g1 = _mm_scale(x, W1, d2)
    acc1 = _sc_msg(g1, src, dst, zrows)
    g2 = _gcn2(acc1[0, :N], acc1[1, :N], g1, d2, b1[None, :], W2)
    acc2 = _sc_msg(g2, src, dst, zrows)
    p0 = _gcn3(acc2[0, :N], acc2[1, :N], g2, d2, b2[None, :],
               Wih0.T, (bih0 + bhh0)[None, :])
    p0pad = jnp.concatenate([p0, jnp.zeros((CSEQ, G4), f32)], axis=0)
    top = jnp.concatenate([Whh0.T, Wih1.T], axis=1)
    bot = jnp.concatenate([jnp.zeros((H, G4), f32), Whh1.T], axis=1)
    m = jnp.concatenate([top, bot], axis=0)
    yspad, h0, c0, h1, c1 = _lstm2(p0pad, m, (bih1 + bhh1)[None, :])
    ys1 = yspad[1:N + 1]

    WcAB = jnp.concatenate([Wc[:H], Wc[H:2 * H]], axis=1)
    sab = _mm_bias(ys1, WcAB, jnp.zeros((1, 2), f32), N, BN, H, 2)
    wef = We @ Wc[2 * H:]
    cconst = (be @ Wc[2 * H:] + bc)[None, :]
    eterm = _mm_bias(edge_features, wef, cconst, E, BE, 16, 1)

    src_out, dst_out, pred = _sc_edges(
        ys1, src, dst, sab[:, 0], sab[:, 1], eterm[:, 0])

    hidden_h = jnp.stack([h0, h1])
    hidden_c = jnp.stack([c0, c1])
    return pred[:, None], hidden_h, hidden_c, src_out, dst_out
